# baseline (speedup 1.0000x reference)
import jax
import jax.numpy as jnp
from jax import lax
from jax.experimental import pallas as pl
from jax.experimental.pallas import tpu as pltpu
from jax.experimental.pallas import tpu_sc as plsc

N_ROWS = 320000
N_FEAT = 128
N_SEG = 10000
NC = 2
NS = 16
NW = NC * NS
SUB = 128
NSLOT = 3
TOTAL_SUB = N_ROWS // SUB
BASE_N = TOTAL_SUB // NW
EXTRA = TOTAL_SUB % NW
SEG_PER_TILE = N_SEG // NS


def _sc_body(x_hbm, idx_hbm, zeros_hbm, part_hbm, acc, xbuf, ibuf, lsem, ssem):
    c = lax.axis_index("c")
    s = lax.axis_index("s")
    wid = c * NS + s

    pltpu.sync_copy(zeros_hbm, acc.at[pl.ds(s * SEG_PER_TILE, SEG_PER_TILE)])
    plsc.subcore_barrier()

    base_sub = wid * BASE_N + jnp.minimum(wid, EXTRA)

    def start_load(g, slot):
        sub = base_sub + g
        pltpu.async_copy(
            x_hbm.at[pl.ds(sub * SUB, SUB)], xbuf.at[slot], lsem.at[slot])
        pltpu.async_copy(
            idx_hbm.at[pl.ds(sub * SUB, SUB)], ibuf.at[slot], lsem.at[slot])

    def wait_load(slot):
        pltpu.make_async_copy(
            x_hbm.at[pl.ds(0, SUB)], xbuf.at[slot], lsem.at[slot]).wait()
        pltpu.make_async_copy(
            idx_hbm.at[pl.ds(0, SUB)], ibuf.at[slot], lsem.at[slot]).wait()

    def start_scatter(slot):
        pltpu.async_copy(
            xbuf.at[slot], acc.at[ibuf.at[slot]], ssem.at[slot], add=True)

    def wait_scatter(slot):
        pltpu.make_async_copy(
            xbuf.at[slot], acc.at[ibuf.at[slot]], ssem.at[slot]).wait()

    start_load(0, 0)
    start_load(1, 1)

    def outer(i, carry):
        for b in range(NSLOT):
            g = i * NSLOT + b
            nxt = (b + 2) % NSLOT
            wait_load(b)
            start_scatter(b)

            @pl.when(g >= 1)
            def _():
                wait_scatter(nxt)

            @pl.when(g + 2 < BASE_N)
            def _():
                start_load(g + 2, nxt)

        return carry

    lax.fori_loop(0, BASE_N // NSLOT, outer, 0)

    wait_scatter((BASE_N - 1) % NSLOT)

    @pl.when(wid < EXTRA)
    def _():
        sub = base_sub + BASE_N
        pltpu.sync_copy(x_hbm.at[pl.ds(sub * SUB, SUB)], xbuf.at[0])
        pltpu.sync_copy(idx_hbm.at[pl.ds(sub * SUB, SUB)], ibuf.at[0])
        pltpu.sync_copy(xbuf.at[0], acc.at[ibuf.at[0]], add=True)

    plsc.subcore_barrier()
    pltpu.sync_copy(
        acc.at[pl.ds(s * SEG_PER_TILE, SEG_PER_TILE)],
        part_hbm.at[c, s],
    )


def _add_body(a_ref, b_ref, o_ref):
    o_ref[...] = a_ref[0] + b_ref[0]


def kernel(x, idx):
    idx1d = idx.astype(jnp.int32)
    zeros = jnp.zeros((SEG_PER_TILE, N_FEAT), jnp.float32)

    part = pl.kernel(
        _sc_body,
        out_type=jax.ShapeDtypeStruct((NC, NS, SEG_PER_TILE, N_FEAT),
                                      jnp.float32),
        mesh=plsc.VectorSubcoreMesh(core_axis_name="c", subcore_axis_name="s"),
        scratch_types=[
            pltpu.VMEM_SHARED((N_SEG, N_FEAT), jnp.float32),
            pltpu.VMEM((NSLOT, SUB, N_FEAT), jnp.float32),
            pltpu.VMEM((NSLOT, SUB), jnp.int32),
            pltpu.SemaphoreType.DMA((NSLOT,)),
            pltpu.SemaphoreType.DMA((NSLOT,)),
        ],
    )(x, idx1d, zeros)

    out = pl.pallas_call(
        _add_body,
        grid=(NS,),
        in_specs=[
            pl.BlockSpec((1, 1, SEG_PER_TILE, N_FEAT), lambda i: (0, i, 0, 0)),
            pl.BlockSpec((1, 1, SEG_PER_TILE, N_FEAT), lambda i: (1, i, 0, 0)),
        ],
        out_specs=pl.BlockSpec((1, SEG_PER_TILE, N_FEAT), lambda i: (i, 0, 0)),
        out_shape=jax.ShapeDtypeStruct((NS, SEG_PER_TILE, N_FEAT),
                                       jnp.float32),
    )(part, part)
    return out.reshape(N_SEG, N_FEAT)


# speedup vs baseline: 1.0848x; 1.0848x over previous
import jax
import jax.numpy as jnp
from jax import lax
from jax.experimental import pallas as pl
from jax.experimental.pallas import tpu as pltpu
from jax.experimental.pallas import tpu_sc as plsc

N_ROWS = 320000
N_FEAT = 128
HALF = N_FEAT // 2
N_SEG = 10000
NC = 2
NS = 16
SUB = 128
TOTAL_SUB = N_ROWS // SUB
BASE_N = TOTAL_SUB // NS
EXTRA = TOTAL_SUB % NS
SEG_PER_TILE = N_SEG // NS


def _sc_body(x_hbm, idx_hbm, zeros_hbm, out_hbm, acc, xbuf, ibuf, sems):
    c = lax.axis_index("c")
    s = lax.axis_index("s")

    pltpu.sync_copy(zeros_hbm, acc.at[pl.ds(s * SEG_PER_TILE, SEG_PER_TILE)])
    plsc.subcore_barrier()

    base_sub = s * BASE_N + jnp.minimum(s, EXTRA)

    def start_load(g, slot):
        sub = base_sub + g
        pltpu.async_copy(
            x_hbm.at[pl.ds(sub * SUB, SUB), pl.ds(c * HALF, HALF)],
            xbuf.at[slot], sems.at[slot])
        pltpu.async_copy(
            idx_hbm.at[pl.ds(sub * SUB, SUB)], ibuf.at[slot], sems.at[slot])

    def wait_load(slot):
        pltpu.make_async_copy(
            x_hbm.at[pl.ds(0, SUB), pl.ds(0, HALF)],
            xbuf.at[slot], sems.at[slot]).wait()
        pltpu.make_async_copy(
            idx_hbm.at[pl.ds(0, SUB)], ibuf.at[slot], sems.at[slot]).wait()

    start_load(0, 0)
    start_load(1, 1)

    def outer(gbase, carry):
        for b in range(2):
            g = gbase + b
            wait_load(b)
            pltpu.sync_copy(xbuf.at[b], acc.at[ibuf.at[b]], add=True)

            @pl.when(g + 2 < BASE_N)
            def _():
                start_load(g + 2, b)

        return carry

    lax.fori_loop(0, BASE_N // 2, lambda i, cr: outer(i * 2, cr), 0)

    @pl.when(s < EXTRA)
    def _():
        sub = base_sub + BASE_N
        pltpu.sync_copy(
            x_hbm.at[pl.ds(sub * SUB, SUB), pl.ds(c * HALF, HALF)],
            xbuf.at[0])
        pltpu.sync_copy(idx_hbm.at[pl.ds(sub * SUB, SUB)], ibuf.at[0])
        pltpu.sync_copy(xbuf.at[0], acc.at[ibuf.at[0]], add=True)

    plsc.subcore_barrier()
    pltpu.sync_copy(
        acc.at[pl.ds(s * SEG_PER_TILE, SEG_PER_TILE)],
        out_hbm.at[pl.ds(s * SEG_PER_TILE, SEG_PER_TILE),
                   pl.ds(c * HALF, HALF)],
    )


def kernel(x, idx):
    idx1d = idx.astype(jnp.int32)
    zeros = jnp.zeros((SEG_PER_TILE, HALF), jnp.float32)

    out = pl.kernel(
        _sc_body,
        out_type=jax.ShapeDtypeStruct((N_SEG, N_FEAT), jnp.float32),
        mesh=plsc.VectorSubcoreMesh(core_axis_name="c", subcore_axis_name="s"),
        compiler_params=pltpu.CompilerParams(use_tc_tiling_on_sc=False),
        scratch_types=[
            pltpu.VMEM_SHARED((N_SEG, HALF), jnp.float32),
            pltpu.VMEM((2, SUB, HALF), jnp.float32),
            pltpu.VMEM((2, SUB), jnp.int32),
            pltpu.SemaphoreType.DMA((2,)),
        ],
    )(x, idx1d, zeros)
    return out


# speedup vs baseline: 1.1599x; 1.0693x over previous
import jax
import jax.numpy as jnp
from jax import lax
from jax.experimental import pallas as pl
from jax.experimental.pallas import tpu as pltpu
from jax.experimental.pallas import tpu_sc as plsc

N_ROWS = 320000
N_FEAT = 128
HALF = N_FEAT // 2
N_SEG = 10000
NC = 2
NS = 16
SUB = 128
G = 4
GROWS = G * SUB
TOTAL_SUB = N_ROWS // SUB
BASE_N = TOTAL_SUB // NS
EXTRA = TOTAL_SUB % NS
GROUPS = BASE_N // G
SEG_PER_TILE = N_SEG // NS


def _sc_body(x_hbm, idx2_hbm, zeros_hbm, out_hbm, acc, xbuf, iall, lsem,
             ssem):
    c = lax.axis_index("c")
    s = lax.axis_index("s")
    base_sub = s * BASE_N + jnp.minimum(s, EXTRA)

    pltpu.sync_copy(idx2_hbm.at[pl.ds(base_sub, BASE_N)],
                    iall.at[pl.ds(0, BASE_N)])

    @pl.when(s < EXTRA)
    def _():
        pltpu.sync_copy(idx2_hbm.at[pl.ds(base_sub + BASE_N, 1)],
                        iall.at[pl.ds(BASE_N, 1)])

    pltpu.sync_copy(zeros_hbm, acc.at[pl.ds(s * SEG_PER_TILE, SEG_PER_TILE)])
    plsc.subcore_barrier()

    def start_load(g, slot):
        sub = base_sub + g * G
        pltpu.async_copy(
            x_hbm.at[pl.ds(sub * SUB, GROWS), pl.ds(c * HALF, HALF)],
            xbuf.at[slot], lsem.at[slot])

    def wait_load(slot):
        pltpu.make_async_copy(
            x_hbm.at[pl.ds(0, GROWS), pl.ds(0, HALF)],
            xbuf.at[slot], lsem.at[slot]).wait()

    def scatter_group(g, slot):
        for k in range(G):
            pltpu.async_copy(
                xbuf.at[slot, pl.ds(k * SUB, SUB)],
                acc.at[iall.at[g * G + k]], ssem.at[slot], add=True)
        for k in range(G):
            pltpu.make_async_copy(
                xbuf.at[slot, pl.ds(k * SUB, SUB)],
                acc.at[iall.at[g * G + k]], ssem.at[slot]).wait()

    start_load(0, 0)
    start_load(1, 1)

    def outer(gbase, carry):
        for b in range(2):
            g = gbase + b
            wait_load(b)
            scatter_group(g, b)

            @pl.when(g + 2 < GROUPS)
            def _():
                start_load(g + 2, b)

        return carry

    lax.fori_loop(0, GROUPS // 2, lambda i, cr: outer(i * 2, cr), 0)

    wait_load((GROUPS - 1) % 2)
    scatter_group(GROUPS - 1, (GROUPS - 1) % 2)

    @pl.when(s < EXTRA)
    def _():
        sub = base_sub + BASE_N
        pltpu.sync_copy(
            x_hbm.at[pl.ds(sub * SUB, SUB), pl.ds(c * HALF, HALF)],
            xbuf.at[0, pl.ds(0, SUB)])
        pltpu.sync_copy(xbuf.at[0, pl.ds(0, SUB)],
                        acc.at[iall.at[BASE_N]], add=True)

    plsc.subcore_barrier()
    pltpu.sync_copy(
        acc.at[pl.ds(s * SEG_PER_TILE, SEG_PER_TILE)],
        out_hbm.at[pl.ds(s * SEG_PER_TILE, SEG_PER_TILE),
                   pl.ds(c * HALF, HALF)],
    )


def kernel(x, idx):
    idx2d = idx.astype(jnp.int32).reshape(TOTAL_SUB, SUB)
    zeros = jnp.zeros((SEG_PER_TILE, HALF), jnp.float32)

    out = pl.kernel(
        _sc_body,
        out_type=jax.ShapeDtypeStruct((N_SEG, N_FEAT), jnp.float32),
        mesh=plsc.VectorSubcoreMesh(core_axis_name="c", subcore_axis_name="s"),
        compiler_params=pltpu.CompilerParams(use_tc_tiling_on_sc=False),
        scratch_types=[
            pltpu.VMEM_SHARED((N_SEG, HALF), jnp.float32),
            pltpu.VMEM((2, GROWS, HALF), jnp.float32),
            pltpu.VMEM((BASE_N + 1, SUB), jnp.int32),
            pltpu.SemaphoreType.DMA((2,)),
            pltpu.SemaphoreType.DMA((2,)),
        ],
    )(x, idx2d, zeros)
    return out


# speedup vs baseline: 1.1738x; 1.0120x over previous
import jax
import jax.numpy as jnp
from jax import lax
from jax.experimental import pallas as pl
from jax.experimental.pallas import tpu as pltpu
from jax.experimental.pallas import tpu_sc as plsc

N_ROWS = 320000
N_FEAT = 128
HALF = N_FEAT // 2
N_SEG = 10000
NC = 2
NS = 16
SUB = 128
G = 2
GROWS = G * SUB
NSLOT = 4
TOTAL_SUB = N_ROWS // SUB
BASE_N = TOTAL_SUB // NS
EXTRA = TOTAL_SUB % NS
GROUPS = BASE_N // G
SEG_PER_TILE = N_SEG // NS


def _sc_body(x_hbm, idx2_hbm, zeros_hbm, out_hbm, acc, xbuf, iall, lsem,
             ssem):
    c = lax.axis_index("c")
    s = lax.axis_index("s")
    base_sub = s * BASE_N + jnp.minimum(s, EXTRA)

    def start_load(g, slot):
        sub = base_sub + g * G
        pltpu.async_copy(
            x_hbm.at[pl.ds(sub * SUB, GROWS), pl.ds(c * HALF, HALF)],
            xbuf.at[slot], lsem.at[slot])

    def wait_load(slot):
        pltpu.make_async_copy(
            x_hbm.at[pl.ds(0, GROWS), pl.ds(0, HALF)],
            xbuf.at[slot], lsem.at[slot]).wait()

    def fire_scatters(g, slot):
        for k in range(G):
            pltpu.async_copy(
                xbuf.at[slot, pl.ds(k * SUB, SUB)],
                acc.at[iall.at[g * G + k]], ssem.at[slot], add=True)

    def drain_scatters(slot):
        for k in range(G):
            pltpu.make_async_copy(
                xbuf.at[slot, pl.ds(k * SUB, SUB)],
                acc.at[iall.at[0]], ssem.at[slot]).wait()

    for p in range(2):
        start_load(p, p)

    pltpu.sync_copy(idx2_hbm.at[pl.ds(base_sub, BASE_N)],
                    iall.at[pl.ds(0, BASE_N)])

    @pl.when(s < EXTRA)
    def _():
        pltpu.sync_copy(idx2_hbm.at[pl.ds(base_sub + BASE_N, 1)],
                        iall.at[pl.ds(BASE_N, 1)])

    pltpu.sync_copy(zeros_hbm, acc.at[pl.ds(s * SEG_PER_TILE, SEG_PER_TILE)])
    plsc.subcore_barrier()

    def step(g, b):
        wait_load(b)
        fire_scatters(g, b)
        s2 = (b + 2) % NSLOT

        @pl.when(g >= 2)
        def _():
            drain_scatters(s2)

        @pl.when(g + 2 < GROUPS)
        def _():
            start_load(g + 2, s2)

    def outer(gbase, carry):
        for b in range(NSLOT):
            step(gbase + b, b)
        return carry

    lax.fori_loop(0, GROUPS // NSLOT, lambda i, cr: outer(i * NSLOT, cr), 0)

    for r in range(GROUPS - GROUPS // NSLOT * NSLOT):
        g = GROUPS // NSLOT * NSLOT + r
        step(g, g % NSLOT)

    drain_scatters((GROUPS - 2) % NSLOT)
    drain_scatters((GROUPS - 1) % NSLOT)

    @pl.when(s < EXTRA)
    def _():
        sub = base_sub + BASE_N
        pltpu.sync_copy(
            x_hbm.at[pl.ds(sub * SUB, SUB), pl.ds(c * HALF, HALF)],
            xbuf.at[0, pl.ds(0, SUB)])
        pltpu.sync_copy(xbuf.at[0, pl.ds(0, SUB)],
                        acc.at[iall.at[BASE_N]], add=True)

    plsc.subcore_barrier()
    pltpu.sync_copy(
        acc.at[pl.ds(s * SEG_PER_TILE, SEG_PER_TILE)],
        out_hbm.at[pl.ds(s * SEG_PER_TILE, SEG_PER_TILE),
                   pl.ds(c * HALF, HALF)],
    )


def kernel(x, idx):
    idx2d = idx.astype(jnp.int32).reshape(TOTAL_SUB, SUB)
    zeros = jnp.zeros((SEG_PER_TILE, HALF), jnp.float32)

    out = pl.kernel(
        _sc_body,
        out_type=jax.ShapeDtypeStruct((N_SEG, N_FEAT), jnp.float32),
        mesh=plsc.VectorSubcoreMesh(core_axis_name="c", subcore_axis_name="s"),
        compiler_params=pltpu.CompilerParams(use_tc_tiling_on_sc=False),
        scratch_types=[
            pltpu.VMEM_SHARED((N_SEG, HALF), jnp.float32),
            pltpu.VMEM((NSLOT, GROWS, HALF), jnp.float32),
            pltpu.VMEM((BASE_N + 1, SUB), jnp.int32),
            pltpu.SemaphoreType.DMA((NSLOT,)),
            pltpu.SemaphoreType.DMA((NSLOT,)),
        ],
    )(x, idx2d, zeros)
    return out
